# baseline (speedup 1.0000x reference)
import functools

import jax
import jax.numpy as jnp
from jax import lax
from jax.experimental import pallas as pl
from jax.experimental.pallas import tpu as pltpu
from jax.experimental.pallas import tpu_sc as plsc

_VOCAB = 1024
_EMBED = 256
_ZCH = 384
_NUM_GROUPS = 32
_GSIZE = _ZCH // _NUM_GROUPS
_EPS = 1e-6
_HW = 1024


def _encode_body(x_ref, m_ref, d_ref, scale_ref, bias_ref, pre_w_ref, cb_ref,
                 cn_ref, z_ref, tok_ref):
    xb = x_ref[0].reshape(_ZCH, _HW)
    xn = (xb - m_ref[0]) / d_ref[0]
    xhat = xn * scale_ref[...] + bias_ref[...]
    z = jnp.dot(pre_w_ref[...], xhat, preferred_element_type=jnp.float32)
    z_ref[0] = z.reshape(_EMBED, 32, 32)
    cb = cb_ref[...]
    zn = jnp.sum(z * z, axis=0, keepdims=True)
    mm = jnp.dot(cb, z, preferred_element_type=jnp.float32)
    dist = (zn + cn_ref[...]) - 2.0 * mm
    vmin = jnp.min(dist, axis=0, keepdims=True)
    vid = lax.broadcasted_iota(jnp.int32, (_VOCAB, _HW), 0)
    idx = jnp.min(jnp.where(dist == vmin, vid, _VOCAB), axis=0)
    tok_ref[0, 0] = idx.astype(jnp.int32)


def _encode(x3, mean_c, d_c, gn_scale, gn_bias, pre_w, codebook, cn2):
    b = x3.shape[0]
    return pl.pallas_call(
        _encode_body,
        grid=(b,),
        in_specs=[
            pl.BlockSpec((1, _ZCH, 32, 32), lambda i: (i, 0, 0, 0)),
            pl.BlockSpec((1, _ZCH, 1), lambda i: (i, 0, 0)),
            pl.BlockSpec((1, _ZCH, 1), lambda i: (i, 0, 0)),
            pl.BlockSpec((_ZCH, 1), lambda i: (0, 0)),
            pl.BlockSpec((_ZCH, 1), lambda i: (0, 0)),
            pl.BlockSpec((_EMBED, _ZCH), lambda i: (0, 0)),
            pl.BlockSpec((_VOCAB, _EMBED), lambda i: (0, 0)),
            pl.BlockSpec((_VOCAB, 1), lambda i: (0, 0)),
        ],
        out_specs=[
            pl.BlockSpec((1, _EMBED, 32, 32), lambda i: (i, 0, 0, 0)),
            pl.BlockSpec((1, 1, _HW), lambda i: (i, 0, 0)),
        ],
        out_shape=[
            jax.ShapeDtypeStruct((b, _EMBED, 32, 32), jnp.float32),
            jax.ShapeDtypeStruct((b, 1, _HW), jnp.int32),
        ],
    )(x3, mean_c, d_c, gn_scale.reshape(_ZCH, 1), gn_bias.reshape(_ZCH, 1),
      pre_w, codebook, cn2)


def _make_sc_gather(n_tokens):
    info = plsc.get_sparse_core_info()
    nw = info.num_cores * info.num_subcores
    bpw = n_tokens // nw
    mesh = plsc.VectorSubcoreMesh(core_axis_name="c", subcore_axis_name="s")

    @functools.partial(
        pl.kernel,
        mesh=mesh,
        out_type=jax.ShapeDtypeStruct((n_tokens, _EMBED), jnp.float32),
        scratch_types=[
            pltpu.VMEM((bpw,), jnp.int32),
            pltpu.VMEM((bpw, _EMBED), jnp.float32),
            pltpu.SemaphoreType.DMA,
        ],
    )
    def gather(cb_hbm, idx_hbm, out_hbm, idx_v, rows_v, sem):
        wid = lax.axis_index("s") * info.num_cores + lax.axis_index("c")
        base = wid * bpw
        pltpu.sync_copy(idx_hbm.at[pl.ds(base, bpw)], idx_v)
        pltpu.async_copy(cb_hbm.at[idx_v], rows_v, sem).wait()
        pltpu.sync_copy(rows_v, out_hbm.at[pl.ds(base, bpw)])

    return gather


def _decode_body(zq_ref, post_w_ref, post_b_ref, zq_out_ref, rec_ref):
    zq_t = zq_ref[0]
    ident = (
        lax.broadcasted_iota(jnp.int32, (_EMBED, _EMBED), 0)
        == lax.broadcasted_iota(jnp.int32, (_EMBED, _EMBED), 1)
    ).astype(jnp.float32)
    zq_cm = lax.dot_general(
        ident, zq_t, (((1,), (1,)), ((), ())),
        preferred_element_type=jnp.float32, precision=lax.Precision.HIGHEST,
    )
    zq_out_ref[0] = zq_cm.reshape(_EMBED, 32, 32)
    rec = lax.dot_general(
        post_w_ref[...], zq_t, (((1,), (1,)), ((), ())),
        preferred_element_type=jnp.float32,
    )
    rec_ref[0] = (rec + post_b_ref[...]).reshape(_ZCH, 32, 32)


def _decode(zq_flat3, post_w, post_b):
    b = zq_flat3.shape[0]
    return pl.pallas_call(
        _decode_body,
        grid=(b,),
        in_specs=[
            pl.BlockSpec((1, _HW, _EMBED), lambda i: (i, 0, 0)),
            pl.BlockSpec((_ZCH, _EMBED), lambda i: (0, 0)),
            pl.BlockSpec((_ZCH, 1), lambda i: (0, 0)),
        ],
        out_specs=[
            pl.BlockSpec((1, _EMBED, 32, 32), lambda i: (i, 0, 0, 0)),
            pl.BlockSpec((1, _ZCH, 32, 32), lambda i: (i, 0, 0, 0)),
        ],
        out_shape=[
            jax.ShapeDtypeStruct((b, _EMBED, 32, 32), jnp.float32),
            jax.ShapeDtypeStruct((b, _ZCH, 32, 32), jnp.float32),
        ],
    )(zq_flat3, post_w, post_b.reshape(_ZCH, 1))


def kernel(x, gn_scale, gn_bias, pre_w, codebook, post_w, post_b):
    b, c, h, w = x.shape
    hw = h * w

    xg = x.reshape(b, _NUM_GROUPS, _GSIZE, h, w)
    mean = xg.mean(axis=(2, 3, 4))
    d = jnp.sqrt(xg.var(axis=(2, 3, 4)) + _EPS)
    mean_c = jnp.repeat(mean, _GSIZE, axis=1).reshape(b, c, 1)
    d_c = jnp.repeat(d, _GSIZE, axis=1).reshape(b, c, 1)

    cn2 = jnp.sum(codebook ** 2, axis=1).reshape(_VOCAB, 1)
    z, tok3 = _encode(x, mean_c, d_c, gn_scale, gn_bias, pre_w, codebook,
                      cn2)
    tokens_flat = tok3.reshape(b * hw)
    zq_flat = _make_sc_gather(b * hw)(codebook, tokens_flat)
    z_q, rec = _decode(zq_flat.reshape(b, hw, _EMBED), post_w, post_b)

    tokens = tok3.reshape(b, hw)
    return (z, z_q, tokens, rec)


# speedup vs baseline: 2.3853x; 2.3853x over previous
import functools

import jax
import jax.numpy as jnp
from jax import lax
from jax.experimental import pallas as pl
from jax.experimental.pallas import tpu as pltpu
from jax.experimental.pallas import tpu_sc as plsc

_VOCAB = 1024
_EMBED = 256
_ZCH = 384
_NUM_GROUPS = 32
_GSIZE = _ZCH // _NUM_GROUPS
_EPS = 1e-6
_HW = 1024


def _encode_body(x_ref, m_ref, d_ref, scale_ref, bias_ref, pre_w_ref, cb_ref,
                 cn_ref, z_ref, tok_ref):
    xb = x_ref[0]
    xn = (xb - m_ref[0]) / d_ref[0]
    xhat = xn * scale_ref[...] + bias_ref[...]
    z = jnp.dot(pre_w_ref[...], xhat, preferred_element_type=jnp.float32)
    z_ref[0] = z
    cb = cb_ref[...]
    zn = jnp.sum(z * z, axis=0, keepdims=True)
    mm = jnp.dot(cb, z, preferred_element_type=jnp.float32)
    dist = (zn + cn_ref[...]) - 2.0 * mm
    vmin = jnp.min(dist, axis=0, keepdims=True)
    vid = lax.broadcasted_iota(jnp.int32, (_VOCAB, _HW), 0)
    idx = jnp.min(jnp.where(dist == vmin, vid, _VOCAB), axis=0)
    tok_ref[0, 0] = idx.astype(jnp.int32)


def _encode(x3, mean_c, d_c, gn_scale, gn_bias, pre_w, codebook, cn2):
    b = x3.shape[0]
    return pl.pallas_call(
        _encode_body,
        grid=(b,),
        in_specs=[
            pl.BlockSpec((1, _ZCH, _HW), lambda i: (i, 0, 0)),
            pl.BlockSpec((1, _ZCH, 1), lambda i: (i, 0, 0)),
            pl.BlockSpec((1, _ZCH, 1), lambda i: (i, 0, 0)),
            pl.BlockSpec((_ZCH, 1), lambda i: (0, 0)),
            pl.BlockSpec((_ZCH, 1), lambda i: (0, 0)),
            pl.BlockSpec((_EMBED, _ZCH), lambda i: (0, 0)),
            pl.BlockSpec((_VOCAB, _EMBED), lambda i: (0, 0)),
            pl.BlockSpec((_VOCAB, 1), lambda i: (0, 0)),
        ],
        out_specs=[
            pl.BlockSpec((1, _EMBED, _HW), lambda i: (i, 0, 0)),
            pl.BlockSpec((1, 1, _HW), lambda i: (i, 0, 0)),
        ],
        out_shape=[
            jax.ShapeDtypeStruct((b, _EMBED, _HW), jnp.float32),
            jax.ShapeDtypeStruct((b, 1, _HW), jnp.int32),
        ],
    )(x3, mean_c, d_c, gn_scale.reshape(_ZCH, 1), gn_bias.reshape(_ZCH, 1),
      pre_w, codebook, cn2)


def _make_sc_gather(n_tokens):
    info = plsc.get_sparse_core_info()
    nw = info.num_cores * info.num_subcores
    bpw = n_tokens // nw
    mesh = plsc.VectorSubcoreMesh(core_axis_name="c", subcore_axis_name="s")

    @functools.partial(
        pl.kernel,
        mesh=mesh,
        out_type=jax.ShapeDtypeStruct((n_tokens, _EMBED), jnp.float32),
        scratch_types=[
            pltpu.VMEM((bpw,), jnp.int32),
            pltpu.VMEM((bpw, _EMBED), jnp.float32),
            pltpu.SemaphoreType.DMA,
        ],
    )
    def gather(cb_hbm, idx_hbm, out_hbm, idx_v, rows_v, sem):
        wid = lax.axis_index("s") * info.num_cores + lax.axis_index("c")
        base = wid * bpw
        pltpu.sync_copy(idx_hbm.at[pl.ds(base, bpw)], idx_v)
        pltpu.async_copy(cb_hbm.at[idx_v], rows_v, sem).wait()
        pltpu.sync_copy(rows_v, out_hbm.at[pl.ds(base, bpw)])

    return gather


def _decode_body(zq_ref, post_w_ref, post_b_ref, rec_ref):
    zq_t = zq_ref[0]
    rec = lax.dot_general(
        post_w_ref[...], zq_t, (((1,), (1,)), ((), ())),
        preferred_element_type=jnp.float32,
    )
    rec_ref[0] = rec + post_b_ref[...]


def _decode(zq_flat3, post_w, post_b):
    b = zq_flat3.shape[0]
    return pl.pallas_call(
        _decode_body,
        grid=(b,),
        in_specs=[
            pl.BlockSpec((1, _HW, _EMBED), lambda i: (i, 0, 0)),
            pl.BlockSpec((_ZCH, _EMBED), lambda i: (0, 0)),
            pl.BlockSpec((_ZCH, 1), lambda i: (0, 0)),
        ],
        out_specs=pl.BlockSpec((1, _ZCH, _HW), lambda i: (i, 0, 0)),
        out_shape=jax.ShapeDtypeStruct((b, _ZCH, _HW), jnp.float32),
    )(zq_flat3, post_w, post_b.reshape(_ZCH, 1))


def kernel(x, gn_scale, gn_bias, pre_w, codebook, post_w, post_b):
    b, c, h, w = x.shape
    hw = h * w
    x3 = x.reshape(b, c, hw)

    xg = x3.reshape(b, _NUM_GROUPS, _GSIZE, hw)
    mean = xg.mean(axis=(2, 3))
    d = jnp.sqrt(xg.var(axis=(2, 3)) + _EPS)
    mean_c = jnp.repeat(mean, _GSIZE, axis=1).reshape(b, c, 1)
    d_c = jnp.repeat(d, _GSIZE, axis=1).reshape(b, c, 1)

    cn2 = jnp.sum(codebook ** 2, axis=1).reshape(_VOCAB, 1)
    z3, tok3 = _encode(x3, mean_c, d_c, gn_scale, gn_bias, pre_w, codebook,
                       cn2)
    tokens_flat = tok3.reshape(b * hw)
    zq_flat = _make_sc_gather(b * hw)(codebook, tokens_flat)
    rec3 = _decode(zq_flat.reshape(b, hw, _EMBED), post_w, post_b)

    z = z3.reshape(b, _EMBED, h, w)
    z_q = jnp.transpose(zq_flat.reshape(b, hw, _EMBED), (0, 2, 1)).reshape(
        b, _EMBED, h, w)
    rec = rec3.reshape(b, c, h, w)
    tokens = tok3.reshape(b, hw)
    return (z, z_q, tokens, rec)
